# baseline (speedup 1.0000x reference)
import functools

import jax
import jax.numpy as jnp
from jax.experimental import pallas as pl

_BM = 4096
_KC = 128
_S = 8


def _lex_min(v_lo, i_lo, v_hi, i_hi):
    take_hi = (v_hi < v_lo) | ((v_hi == v_lo) & (i_hi < i_lo))
    return jnp.where(take_hi, v_hi, v_lo), jnp.where(take_hi, i_hi, i_lo)


def _vq_argmin_kernel(z_ref, cb_ref, out_ref):
    z = z_ref[...]
    cb = cb_ref[...]
    bm = z.shape[0]
    k = cb.shape[0]
    zsq = jnp.sum(z * z, axis=1, keepdims=True)
    csq = jnp.sum(cb * cb, axis=1)
    cb2 = cb + cb
    lane_iota = jax.lax.broadcasted_iota(jnp.int32, (bm, _KC), 1)
    m = None
    idx = None
    for c in range(k // _KC):
        dots2 = jax.lax.dot_general(
            z, cb2[c * _KC:(c + 1) * _KC, :], (((1,), (1,)), ((), ())),
            preferred_element_type=jnp.float32)
        dist = zsq + csq[c * _KC:(c + 1) * _KC] - dots2
        if m is None:
            m = dist
            idx = lane_iota
        else:
            pred = dist < m
            m = jnp.where(pred, dist, m)
            idx = jnp.where(pred, lane_iota + c * _KC, idx)
    mt = m.T
    it = idx.T
    v = mt[0:_S, :]
    ix = it[0:_S, :]
    for s in range(1, _KC // _S):
        v, ix = _lex_min(v, ix, mt[s * _S:(s + 1) * _S, :],
                         it[s * _S:(s + 1) * _S, :])
    h = _S
    while h > 1:
        h //= 2
        v, ix = _lex_min(v[:h, :], ix[:h, :], v[h:2 * h, :], ix[h:2 * h, :])
    out_ref[...] = ix[0]


@functools.partial(jax.jit, static_argnames=())
def kernel(z, W_dec, codebook):
    del W_dec
    n, d = z.shape
    k = codebook.shape[0]
    grid = n // _BM
    return pl.pallas_call(
        _vq_argmin_kernel,
        grid=(grid,),
        in_specs=[
            pl.BlockSpec((_BM, d), lambda i: (i, 0)),
            pl.BlockSpec((k, d), lambda i: (0, 0)),
        ],
        out_specs=pl.BlockSpec((_BM,), lambda i: (i,)),
        out_shape=jax.ShapeDtypeStruct((n,), jnp.int32),
    )(z, codebook)


# speedup vs baseline: 1.1417x; 1.1417x over previous
import functools

import jax
import jax.numpy as jnp
from jax.experimental import pallas as pl
from jax.experimental.pallas import tpu as pltpu

_BM = 4096


_S = 8


def _vq_argmin_kernel(z_ref, cb_ref, out_ref):
    z = z_ref[...]
    zt = z
    cb = cb_ref[...]
    bm = z.shape[0]
    k = cb.shape[0]
    zsq = jnp.sum(z * z, axis=1, keepdims=True).T
    csq = jnp.sum(cb * cb, axis=1, keepdims=True)
    cb2 = cb + cb
    kc = 128
    sub_iota = jax.lax.broadcasted_iota(jnp.int32, (_S, bm), 0)
    m = None
    idx = None
    for c in range(k // kc):
        dots2 = jax.lax.dot_general(
            cb2[c * kc:(c + 1) * kc, :], zt, (((1,), (1,)), ((), ())),
            preferred_element_type=jnp.float32)
        dist = zsq + csq[c * kc:(c + 1) * kc, :] - dots2
        for i in range(kc // _S):
            val = dist[i * _S:(i + 1) * _S, :]
            base = c * kc + i * _S
            if m is None:
                m = val
                idx = sub_iota
            else:
                pred = val < m
                m = jnp.where(pred, val, m)
                idx = jnp.where(pred, sub_iota + base, idx)
    h = _S
    while h > 1:
        h //= 2
        v_lo, v_hi = m[:h, :], m[h:2 * h, :]
        i_lo, i_hi = idx[:h, :], idx[h:2 * h, :]
        take_hi = (v_hi < v_lo) | ((v_hi == v_lo) & (i_hi < i_lo))
        m = jnp.where(take_hi, v_hi, v_lo)
        idx = jnp.where(take_hi, i_hi, i_lo)
    out_ref[...] = idx[0]


@functools.partial(jax.jit, static_argnames=())
def kernel(z, W_dec, codebook):
    del W_dec
    n, d = z.shape
    k = codebook.shape[0]
    grid = n // _BM
    return pl.pallas_call(
        _vq_argmin_kernel,
        grid=(grid,),
        in_specs=[
            pl.BlockSpec((_BM, d), lambda i: (i, 0)),
            pl.BlockSpec((k, d), lambda i: (0, 0)),
        ],
        out_specs=pl.BlockSpec((_BM,), lambda i: (i,)),
        out_shape=jax.ShapeDtypeStruct((n,), jnp.int32),
    )(z, codebook)
